# baseline (speedup 1.0000x reference)
import functools

import jax
import jax.numpy as jnp
from jax.experimental import pallas as pl

_BR = 2048


def _body(op_ref, nte_ref, se_ref, q_ref, pnst_ref, ga_ref, r_ref, gc_ref,
          inds_ref, sc_ref, lp_ref, *, L, T, S, NH):
    f32 = jnp.float32

    ntl = jax.lax.dot_general(q_ref[:], nte_ref[:], (((1,), (1,)), ((), ())),
                              preferred_element_type=f32)
    m0 = jnp.max(ntl, axis=1, keepdims=True)
    sh = ntl - m0
    lp_nt = sh - jnp.log(jnp.sum(jnp.exp(sh), axis=1, keepdims=True))

    ga = ga_ref[:]
    x = ga + ntl
    xm = jnp.max(x, axis=1, keepdims=True)
    io_nh = jax.lax.broadcasted_iota(jnp.int32, x.shape, 1)
    idx = jnp.min(jnp.where(x == xm, io_nh, NH), axis=1, keepdims=True)
    sel_nh = io_nh == idx
    lp_num = jnp.sum(jnp.where(sel_nh, lp_nt, 0.0), axis=1, keepdims=True)
    nt = jnp.sum(jnp.where(sel_nh, pnst_ref[:], 0), axis=1, keepdims=True)

    tbl = jax.lax.dot_general(op_ref[:], se_ref[:], (((1,), (1,)), ((), ())),
                              preferred_element_type=f32)
    tmax = jnp.max(tbl, axis=1, keepdims=True)
    lse = jnp.log(jnp.sum(jnp.exp(tbl - tmax), axis=1, keepdims=True))
    lp_tbl = (tbl - tmax) - lse
    tbl3 = jnp.concatenate([tbl] * L, axis=1)
    lp3 = jnp.concatenate([lp_tbl] * L, axis=1)

    r = r_ref[:]
    io_l = jax.lax.broadcasted_iota(jnp.int32, r.shape, 1)
    mask = io_l >= nt
    inds = jnp.where(mask, 0, r)
    inds_ref[:, :] = inds

    io93 = jax.lax.broadcasted_iota(jnp.int32, (r.shape[0], L * S), 1)
    grp = io93 // S
    ind93 = jnp.where(grp == 0, inds[:, 0:1],
                      jnp.where(grp == 1, inds[:, 1:2], inds[:, 2:3]))
    sel0 = ind93 == 0
    rows = jnp.where(sel0, tbl3[0:1, :], 0.0)
    lpr = jnp.where(sel0, lp3[0:1, :], 0.0)
    for k in range(1, T):
        sel = ind93 == k
        rows = rows + jnp.where(sel, tbl3[k:k + 1, :], 0.0)
        lpr = lpr + jnp.where(sel, lp3[k:k + 1, :], 0.0)

    y = rows + gc_ref[:]

    acc = lp_num
    scs = []
    for l in range(L):
        y_l = y[:, l * S:(l + 1) * S]
        ym = jnp.max(y_l, axis=1, keepdims=True)
        io_s = jax.lax.broadcasted_iota(jnp.int32, y_l.shape, 1)
        sc = jnp.min(jnp.where(y_l == ym, io_s, S), axis=1, keepdims=True)
        scs.append(sc)
        lp_l = jnp.sum(jnp.where(io_s == sc, lpr[:, l * S:(l + 1) * S], 0.0),
                       axis=1, keepdims=True)
        acc = acc + jnp.where(mask[:, l:l + 1], 0.0, lp_l)
    sc_ref[:, :] = jnp.concatenate(scs, axis=1)
    lp_ref[:, :] = acc


def kernel(imgs, op_embs, num_transforms_embs, scale_embs, q, pnst):
    B = imgs.shape[0]
    T = op_embs.shape[0]
    S = scale_embs.shape[0]
    NH = num_transforms_embs.shape[0]
    L = NH - 1
    H = q.shape[0]

    skey = jax.random.key(42)
    kA, kB, kC = jax.random.split(skey, 3)
    gA = jax.random.gumbel(kA, (B, NH), jnp.float32)
    rinds = jax.random.randint(kB, (B, L), 0, T)
    gC = jax.random.gumbel(kC, (B * L, S), jnp.float32)
    g93 = gC.reshape(B, L * S)

    q_r = q.reshape(1, H)
    pnst_r = pnst.reshape(1, NH)

    nblk = B // _BR
    full = lambda *shape: pl.BlockSpec(shape, lambda i: (0,) * len(shape))
    inds, sc, lp = pl.pallas_call(
        functools.partial(_body, L=L, T=T, S=S, NH=NH),
        grid=(nblk,),
        in_specs=[
            full(T, H),
            full(NH, H),
            full(S, H),
            full(1, H),
            full(1, NH),
            pl.BlockSpec((_BR, NH), lambda i: (i, 0)),
            pl.BlockSpec((_BR, L), lambda i: (i, 0)),
            pl.BlockSpec((_BR, L * S), lambda i: (i, 0)),
        ],
        out_specs=[
            pl.BlockSpec((_BR, L), lambda i: (i, 0)),
            pl.BlockSpec((_BR, L), lambda i: (i, 0)),
            pl.BlockSpec((_BR, 1), lambda i: (i, 0)),
        ],
        out_shape=[
            jax.ShapeDtypeStruct((B, L), jnp.int32),
            jax.ShapeDtypeStruct((B, L), jnp.int32),
            jax.ShapeDtypeStruct((B, 1), jnp.float32),
        ],
    )(op_embs, num_transforms_embs, scale_embs, q_r, pnst_r, gA, rinds, g93)

    return (inds, sc, lp.reshape(B))


# speedup vs baseline: 3.6080x; 3.6080x over previous
import functools

import jax
import jax.numpy as jnp
from jax.experimental import pallas as pl

_BB = 2048
_TINY = float(jnp.finfo(jnp.float32).tiny)


def _threefry_bits(k1, k2, cnt):
    ks = (k1, k2, k1 ^ k2 ^ jnp.uint32(0x1BD11BDA))
    rots = ((13, 15, 26, 6), (17, 29, 16, 24))
    x0 = jnp.zeros_like(cnt) + ks[0]
    x1 = cnt + ks[1]
    for g in range(5):
        for r in rots[g % 2]:
            x0 = x0 + x1
            x1 = (x1 << r) | (x1 >> (32 - r))
            x1 = x1 ^ x0
        x0 = x0 + ks[(g + 1) % 3]
        x1 = x1 + ks[(g + 2) % 3] + jnp.uint32(g + 1)
    return x0 ^ x1


def _gumbel(k1, k2, cnt):
    bits = _threefry_bits(k1, k2, cnt)
    fb = (bits >> jnp.uint32(9)) | jnp.uint32(0x3F800000)
    fl = jax.lax.bitcast_convert_type(fb, jnp.float32) - jnp.float32(1.0)
    u = jnp.maximum(jnp.float32(_TINY),
                    fl * jnp.float32(1.0 - _TINY) + jnp.float32(_TINY))
    return -jnp.log(-jnp.log(u))


def _body(op_ref, nte_ref, se_ref, q_ref, pnst_ref, kd_ref,
          inds_ref, sc_ref, lp_ref, *, L, T, S, NH):
    f32 = jnp.float32
    u32 = jnp.uint32
    base = (pl.program_id(0) * _BB).astype(u32)
    bvec = base + jax.lax.broadcasted_iota(u32, (1, _BB), 1)

    ntl = jnp.dot(nte_ref[:], q_ref[:], preferred_element_type=f32)
    m0 = jnp.max(ntl, axis=0, keepdims=True)
    sh = ntl - m0
    lp_nt = sh - jnp.log(jnp.sum(jnp.exp(sh), axis=0, keepdims=True))

    jio = jax.lax.broadcasted_iota(u32, (NH, _BB), 0)
    ga = _gumbel(kd_ref[0, 0], kd_ref[0, 1], u32(NH) * bvec + jio)
    x = ga + ntl
    xm = jnp.max(x, axis=0, keepdims=True)
    io_nh = jax.lax.broadcasted_iota(jnp.int32, x.shape, 0)
    idx = jnp.min(jnp.where(x == xm, io_nh, NH), axis=0, keepdims=True)
    sel_nh = io_nh == idx
    lp_num = jnp.sum(jnp.where(sel_nh, lp_nt, 0.0), axis=0, keepdims=True)
    nt = jnp.sum(jnp.where(sel_nh, pnst_ref[:], 0), axis=0, keepdims=True)

    lio = jax.lax.broadcasted_iota(u32, (L, _BB), 0)
    rbits = _threefry_bits(kd_ref[1, 0], kd_ref[1, 1], u32(L) * bvec + lio)
    mul = ((1 << 16) % T) ** 2 % T
    r = ((rbits >> 16) % u32(T) * u32(mul) + (rbits & u32(0xFFFF)) % u32(T))
    r = (r % u32(T)).astype(jnp.int32)

    tblT = jax.lax.dot_general(se_ref[:], op_ref[:], (((1,), (1,)), ((), ())),
                               preferred_element_type=f32)
    tmax = jnp.max(tblT, axis=0, keepdims=True)
    lse = jnp.log(jnp.sum(jnp.exp(tblT - tmax), axis=0, keepdims=True))
    c_row = tmax + lse

    io_s = jax.lax.broadcasted_iota(jnp.int32, (S, _BB), 0)
    sio = jax.lax.broadcasted_iota(u32, (S, _BB), 0)
    cbase = u32(L * S) * bvec
    acc = lp_num
    for l in range(L):
        mask_l = nt <= l
        ind_l = jnp.where(mask_l, 0, r[l:l + 1, :])
        inds_ref[l:l + 1, :] = ind_l
        rows = jnp.where(ind_l == 0, tblT[:, 0:1], 0.0)
        logz = jnp.where(ind_l == 0, c_row[:, 0:1], 0.0)
        for k in range(1, T):
            sel = ind_l == k
            rows = rows + jnp.where(sel, tblT[:, k:k + 1], 0.0)
            logz = logz + jnp.where(sel, c_row[:, k:k + 1], 0.0)
        gc_l = _gumbel(kd_ref[2, 0], kd_ref[2, 1], cbase + u32(S * l) + sio)
        y = rows + gc_l
        ym = jnp.max(y, axis=0, keepdims=True)
        sc = jnp.min(jnp.where(y == ym, io_s, S), axis=0, keepdims=True)
        sc_ref[l:l + 1, :] = sc
        chosen = jnp.sum(jnp.where(io_s == sc, rows, 0.0), axis=0, keepdims=True)
        acc = acc + jnp.where(mask_l, 0.0, chosen - logz)
    lp_ref[:] = acc


def kernel(imgs, op_embs, num_transforms_embs, scale_embs, q, pnst):
    B = imgs.shape[0]
    T = op_embs.shape[0]
    S = scale_embs.shape[0]
    NH = num_transforms_embs.shape[0]
    L = NH - 1
    H = q.shape[0]

    skey = jax.random.key(42)
    kA, kB, kC = jax.random.split(skey, 3)
    kB2 = jax.random.split(kB, 2)[1]
    kd = jnp.stack([jax.random.key_data(kA), jax.random.key_data(kB2),
                    jax.random.key_data(kC)]).astype(jnp.uint32)

    q_c = q.reshape(H, 1)
    pnst_c = pnst.reshape(NH, 1)

    nblk = B // _BB
    full = lambda *shape: pl.BlockSpec(shape, lambda i: (0,) * len(shape))
    inds_t, sc_t, lp = pl.pallas_call(
        functools.partial(_body, L=L, T=T, S=S, NH=NH),
        grid=(nblk,),
        in_specs=[
            full(T, H),
            full(NH, H),
            full(S, H),
            full(H, 1),
            full(NH, 1),
            full(3, 2),
        ],
        out_specs=[
            pl.BlockSpec((L, _BB), lambda i: (0, i)),
            pl.BlockSpec((L, _BB), lambda i: (0, i)),
            pl.BlockSpec((1, _BB), lambda i: (0, i)),
        ],
        out_shape=[
            jax.ShapeDtypeStruct((L, B), jnp.int32),
            jax.ShapeDtypeStruct((L, B), jnp.int32),
            jax.ShapeDtypeStruct((1, B), jnp.float32),
        ],
    )(op_embs, num_transforms_embs, scale_embs, q_c, pnst_c, kd)

    return (inds_t.T, sc_t.T, lp.reshape(B))


# speedup vs baseline: 4.1808x; 1.1588x over previous
import functools

import jax
import jax.numpy as jnp
from jax.experimental import pallas as pl

_BB = 2048
_TINY = float(jnp.finfo(jnp.float32).tiny)
_KA = (0x6D3E048F, 0x1022172D)
_KB2 = (0x8C1266AC, 0x45A3D6BE)
_KC = (0x92FB20EA, 0x0F38D913)


def _threefry_bits(key, cnt):
    k1, k2 = jnp.uint32(key[0]), jnp.uint32(key[1])
    ks = (k1, k2, k1 ^ k2 ^ jnp.uint32(0x1BD11BDA))
    rots = ((13, 15, 26, 6), (17, 29, 16, 24))
    x0 = jnp.zeros_like(cnt) + ks[0]
    x1 = cnt + ks[1]
    for g in range(5):
        for r in rots[g % 2]:
            x0 = x0 + x1
            x1 = (x1 << r) | (x1 >> (32 - r))
            x1 = x1 ^ x0
        x0 = x0 + ks[(g + 1) % 3]
        x1 = x1 + ks[(g + 2) % 3] + jnp.uint32(g + 1)
    return x0 ^ x1


def _gumbel(key, cnt):
    bits = _threefry_bits(key, cnt)
    fb = (bits >> jnp.uint32(9)) | jnp.uint32(0x3F800000)
    fl = jax.lax.bitcast_convert_type(fb, jnp.float32) - jnp.float32(1.0)
    u = jnp.maximum(jnp.float32(_TINY),
                    fl * jnp.float32(1.0 - _TINY) + jnp.float32(_TINY))
    return -jnp.log(-jnp.log(u))


def _body(op_ref, nte_ref, se_ref, q_ref, pnst_ref,
          inds_ref, sc_ref, lp_ref, *, L, T, S, NH):
    f32 = jnp.float32
    u32 = jnp.uint32
    base = (pl.program_id(0) * _BB).astype(u32)
    bvec = base + jax.lax.broadcasted_iota(u32, (1, _BB), 1)

    ntl = jnp.dot(nte_ref[:], q_ref[:], preferred_element_type=f32)
    m0 = jnp.max(ntl, axis=0, keepdims=True)
    sh = ntl - m0
    lp_nt = sh - jnp.log(jnp.sum(jnp.exp(sh), axis=0, keepdims=True))

    jio = jax.lax.broadcasted_iota(u32, (NH, _BB), 0)
    ga = _gumbel(_KA, u32(NH) * bvec + jio)
    x = ga + ntl
    xm = jnp.max(x, axis=0, keepdims=True)
    io_nh = jax.lax.broadcasted_iota(jnp.int32, x.shape, 0)
    idx = jnp.min(jnp.where(x == xm, io_nh, NH), axis=0, keepdims=True)
    sel_nh = io_nh == idx
    lp_num = jnp.sum(jnp.where(sel_nh, lp_nt, 0.0), axis=0, keepdims=True)
    nt = jnp.sum(jnp.where(sel_nh, pnst_ref[:], 0), axis=0, keepdims=True)

    lio = jax.lax.broadcasted_iota(u32, (L, _BB), 0)
    rbits = _threefry_bits(_KB2, u32(L) * bvec + lio)
    if T & (T - 1) == 0:
        r = (rbits & u32(T - 1)).astype(jnp.int32)
    else:
        mul = ((1 << 16) % T) ** 2 % T
        r = ((rbits >> 16) % u32(T) * u32(mul) + (rbits & u32(0xFFFF)) % u32(T))
        r = (r % u32(T)).astype(jnp.int32)

    tblT = jax.lax.dot_general(se_ref[:], op_ref[:], (((1,), (1,)), ((), ())),
                               preferred_element_type=f32)
    tmax = jnp.max(tblT, axis=0, keepdims=True)
    lse = jnp.log(jnp.sum(jnp.exp(tblT - tmax), axis=0, keepdims=True))
    aug = jnp.concatenate([tblT, tmax + lse], axis=0)

    io_s = jax.lax.broadcasted_iota(jnp.int32, (S, _BB), 0)
    sio = jax.lax.broadcasted_iota(u32, (S, _BB), 0)
    cbase = u32(L * S) * bvec
    acc = lp_num
    ind_cols, sc_cols = [], []
    for l in range(L):
        mask_l = nt <= l
        ind_l = jnp.where(mask_l, 0, r[l:l + 1, :])
        ind_cols.append(ind_l)
        lvl = [aug[:, k:k + 1] for k in range(T)]
        bit = 1
        while len(lvl) > 1:
            cond = (ind_l & bit) != 0
            lvl = [jnp.where(cond, lvl[2 * j + 1], lvl[2 * j])
                   for j in range(len(lvl) // 2)]
            bit <<= 1
        rows = lvl[0][:S, :]
        logz = lvl[0][S:, :]
        gc_l = _gumbel(_KC, cbase + u32(S * l) + sio)
        y = rows + gc_l
        ym = jnp.max(y, axis=0, keepdims=True)
        sc = jnp.min(jnp.where(y == ym, io_s, S), axis=0, keepdims=True)
        sc_cols.append(sc)
        chosen = jnp.sum(jnp.where(io_s == sc, rows, 0.0), axis=0, keepdims=True)
        acc = acc + jnp.where(mask_l, 0.0, chosen - logz)
    inds_ref[:, :] = jnp.concatenate(ind_cols, axis=0).T
    sc_ref[:, :] = jnp.concatenate(sc_cols, axis=0).T
    lp_ref[:, :] = acc.T


def kernel(imgs, op_embs, num_transforms_embs, scale_embs, q, pnst):
    B = imgs.shape[0]
    T = op_embs.shape[0]
    S = scale_embs.shape[0]
    NH = num_transforms_embs.shape[0]
    L = NH - 1
    H = q.shape[0]

    q_c = q.reshape(H, 1)
    pnst_c = pnst.reshape(NH, 1)

    nblk = B // _BB
    full = lambda *shape: pl.BlockSpec(shape, lambda i: (0,) * len(shape))
    inds, sc, lp = pl.pallas_call(
        functools.partial(_body, L=L, T=T, S=S, NH=NH),
        grid=(nblk,),
        in_specs=[
            full(T, H),
            full(NH, H),
            full(S, H),
            full(H, 1),
            full(NH, 1),
        ],
        out_specs=[
            pl.BlockSpec((_BB, L), lambda i: (i, 0)),
            pl.BlockSpec((_BB, L), lambda i: (i, 0)),
            pl.BlockSpec((_BB, 1), lambda i: (i, 0)),
        ],
        out_shape=[
            jax.ShapeDtypeStruct((B, L), jnp.int32),
            jax.ShapeDtypeStruct((B, L), jnp.int32),
            jax.ShapeDtypeStruct((B, 1), jnp.float32),
        ],
    )(op_embs, num_transforms_embs, scale_embs, q_c, pnst_c)

    return (inds, sc, lp.reshape(B))


# speedup vs baseline: 4.2236x; 1.0102x over previous
import functools

import jax
import jax.numpy as jnp
from jax.experimental import pallas as pl

_BB = 4096
_TINY = float(jnp.finfo(jnp.float32).tiny)
_KA = (0x6D3E048F, 0x1022172D)
_KB2 = (0x8C1266AC, 0x45A3D6BE)
_KC = (0x92FB20EA, 0x0F38D913)


def _threefry_bits(key, cnt):
    k1, k2 = jnp.uint32(key[0]), jnp.uint32(key[1])
    ks = (k1, k2, k1 ^ k2 ^ jnp.uint32(0x1BD11BDA))
    rots = ((13, 15, 26, 6), (17, 29, 16, 24))
    x0 = jnp.zeros_like(cnt) + ks[0]
    x1 = cnt + ks[1]
    for g in range(5):
        for r in rots[g % 2]:
            x0 = x0 + x1
            x1 = (x1 << r) | (x1 >> (32 - r))
            x1 = x1 ^ x0
        x0 = x0 + ks[(g + 1) % 3]
        x1 = x1 + ks[(g + 2) % 3] + jnp.uint32(g + 1)
    return x0 ^ x1


def _gumbel(key, cnt):
    bits = _threefry_bits(key, cnt)
    fb = (bits >> jnp.uint32(9)) | jnp.uint32(0x3F800000)
    fl = jax.lax.bitcast_convert_type(fb, jnp.float32) - jnp.float32(1.0)
    u = jnp.maximum(jnp.float32(_TINY),
                    fl * jnp.float32(1.0 - _TINY) + jnp.float32(_TINY))
    return -jnp.log(-jnp.log(u))


def _body(op_ref, nte_ref, se_ref, q_ref, pnst_ref,
          inds_ref, sc_ref, lp_ref, *, L, T, S, NH):
    f32 = jnp.float32
    u32 = jnp.uint32
    base = (pl.program_id(0) * _BB).astype(u32)
    bvec = base + jax.lax.broadcasted_iota(u32, (1, _BB), 1)

    ntl = jnp.dot(nte_ref[:], q_ref[:], preferred_element_type=f32)
    m0 = jnp.max(ntl, axis=0, keepdims=True)
    sh = ntl - m0
    lp_nt = sh - jnp.log(jnp.sum(jnp.exp(sh), axis=0, keepdims=True))

    jio = jax.lax.broadcasted_iota(u32, (NH, _BB), 0)
    ga = _gumbel(_KA, u32(NH) * bvec + jio)
    x = ga + ntl
    xm = jnp.max(x, axis=0, keepdims=True)
    io_nh = jax.lax.broadcasted_iota(jnp.int32, x.shape, 0)
    idx = jnp.min(jnp.where(x == xm, io_nh, NH), axis=0, keepdims=True)
    sel_nh = io_nh == idx
    lp_num = jnp.sum(jnp.where(sel_nh, lp_nt, 0.0), axis=0, keepdims=True)
    nt = jnp.sum(jnp.where(sel_nh, pnst_ref[:], 0), axis=0, keepdims=True)

    lio = jax.lax.broadcasted_iota(u32, (L, _BB), 0)
    rbits = _threefry_bits(_KB2, u32(L) * bvec + lio)
    if T & (T - 1) == 0:
        r = (rbits & u32(T - 1)).astype(jnp.int32)
    else:
        mul = ((1 << 16) % T) ** 2 % T
        r = ((rbits >> 16) % u32(T) * u32(mul) + (rbits & u32(0xFFFF)) % u32(T))
        r = (r % u32(T)).astype(jnp.int32)

    tblT = jax.lax.dot_general(se_ref[:], op_ref[:], (((1,), (1,)), ((), ())),
                               preferred_element_type=f32)
    tmax = jnp.max(tblT, axis=0, keepdims=True)
    lse = jnp.log(jnp.sum(jnp.exp(tblT - tmax), axis=0, keepdims=True))
    aug = jnp.concatenate([tblT, tmax + lse], axis=0)

    io_s = jax.lax.broadcasted_iota(jnp.int32, (S, _BB), 0)
    sio = jax.lax.broadcasted_iota(u32, (S, _BB), 0)
    cbase = u32(L * S) * bvec
    acc = lp_num
    ind_cols, sc_cols = [], []
    for l in range(L):
        mask_l = nt <= l
        ind_l = jnp.where(mask_l, 0, r[l:l + 1, :])
        ind_cols.append(ind_l)
        lvl = [aug[:, k:k + 1] for k in range(T)]
        bit = 1
        while len(lvl) > 1:
            cond = (ind_l & bit) != 0
            lvl = [jnp.where(cond, lvl[2 * j + 1], lvl[2 * j])
                   for j in range(len(lvl) // 2)]
            bit <<= 1
        rows = lvl[0][:S, :]
        logz = lvl[0][S:, :]
        gc_l = _gumbel(_KC, cbase + u32(S * l) + sio)
        y = rows + gc_l
        ym = jnp.max(y, axis=0, keepdims=True)
        sc = jnp.min(jnp.where(y == ym, io_s, S), axis=0, keepdims=True)
        sc_cols.append(sc)
        chosen = jnp.sum(jnp.where(io_s == sc, rows, 0.0), axis=0, keepdims=True)
        acc = acc + jnp.where(mask_l, 0.0, chosen - logz)
    inds_ref[:, :] = jnp.concatenate(ind_cols, axis=0).T
    sc_ref[:, :] = jnp.concatenate(sc_cols, axis=0).T
    lp_ref[:, :] = acc.T


def kernel(imgs, op_embs, num_transforms_embs, scale_embs, q, pnst):
    B = imgs.shape[0]
    T = op_embs.shape[0]
    S = scale_embs.shape[0]
    NH = num_transforms_embs.shape[0]
    L = NH - 1
    H = q.shape[0]

    q_c = q.reshape(H, 1)
    pnst_c = pnst.reshape(NH, 1)

    nblk = B // _BB
    full = lambda *shape: pl.BlockSpec(shape, lambda i: (0,) * len(shape))
    inds, sc, lp = pl.pallas_call(
        functools.partial(_body, L=L, T=T, S=S, NH=NH),
        grid=(nblk,),
        in_specs=[
            full(T, H),
            full(NH, H),
            full(S, H),
            full(H, 1),
            full(NH, 1),
        ],
        out_specs=[
            pl.BlockSpec((_BB, L), lambda i: (i, 0)),
            pl.BlockSpec((_BB, L), lambda i: (i, 0)),
            pl.BlockSpec((_BB, 1), lambda i: (i, 0)),
        ],
        out_shape=[
            jax.ShapeDtypeStruct((B, L), jnp.int32),
            jax.ShapeDtypeStruct((B, L), jnp.int32),
            jax.ShapeDtypeStruct((B, 1), jnp.float32),
        ],
    )(op_embs, num_transforms_embs, scale_embs, q_c, pnst_c)

    return (inds, sc, lp.reshape(B))


# speedup vs baseline: 4.2439x; 1.0048x over previous
import functools

import jax
import jax.numpy as jnp
from jax.experimental import pallas as pl

_BB = 8192
_TINY = float(jnp.finfo(jnp.float32).tiny)
_KA = (0x6D3E048F, 0x1022172D)
_KB2 = (0x8C1266AC, 0x45A3D6BE)
_KC = (0x92FB20EA, 0x0F38D913)


def _threefry_bits(key, cnt):
    k1, k2 = jnp.uint32(key[0]), jnp.uint32(key[1])
    ks = (k1, k2, k1 ^ k2 ^ jnp.uint32(0x1BD11BDA))
    rots = ((13, 15, 26, 6), (17, 29, 16, 24))
    x0 = jnp.zeros_like(cnt) + ks[0]
    x1 = cnt + ks[1]
    for g in range(5):
        for r in rots[g % 2]:
            x0 = x0 + x1
            x1 = (x1 << r) | (x1 >> (32 - r))
            x1 = x1 ^ x0
        x0 = x0 + ks[(g + 1) % 3]
        x1 = x1 + ks[(g + 2) % 3] + jnp.uint32(g + 1)
    return x0 ^ x1


def _gumbel(key, cnt):
    bits = _threefry_bits(key, cnt)
    fb = (bits >> jnp.uint32(9)) | jnp.uint32(0x3F800000)
    fl = jax.lax.bitcast_convert_type(fb, jnp.float32) - jnp.float32(1.0)
    u = jnp.maximum(jnp.float32(_TINY),
                    fl * jnp.float32(1.0 - _TINY) + jnp.float32(_TINY))
    return -jnp.log(-jnp.log(u))


def _body(op_ref, nte_ref, se_ref, q_ref, pnst_ref,
          inds_ref, sc_ref, lp_ref, *, L, T, S, NH):
    f32 = jnp.float32
    u32 = jnp.uint32
    base = (pl.program_id(0) * _BB).astype(u32)
    bvec = base + jax.lax.broadcasted_iota(u32, (1, _BB), 1)

    ntl = jnp.dot(nte_ref[:], q_ref[:], preferred_element_type=f32)
    m0 = jnp.max(ntl, axis=0, keepdims=True)
    sh = ntl - m0
    lp_nt = sh - jnp.log(jnp.sum(jnp.exp(sh), axis=0, keepdims=True))

    jio = jax.lax.broadcasted_iota(u32, (NH, _BB), 0)
    ga = _gumbel(_KA, u32(NH) * bvec + jio)
    x = ga + ntl
    xm = jnp.max(x, axis=0, keepdims=True)
    io_nh = jax.lax.broadcasted_iota(jnp.int32, x.shape, 0)
    idx = jnp.min(jnp.where(x == xm, io_nh, NH), axis=0, keepdims=True)
    sel_nh = io_nh == idx
    lp_num = jnp.sum(jnp.where(sel_nh, lp_nt, 0.0), axis=0, keepdims=True)
    nt = jnp.sum(jnp.where(sel_nh, pnst_ref[:], 0), axis=0, keepdims=True)

    lio = jax.lax.broadcasted_iota(u32, (L, _BB), 0)
    rbits = _threefry_bits(_KB2, u32(L) * bvec + lio)
    if T & (T - 1) == 0:
        r = (rbits & u32(T - 1)).astype(jnp.int32)
    else:
        mul = ((1 << 16) % T) ** 2 % T
        r = ((rbits >> 16) % u32(T) * u32(mul) + (rbits & u32(0xFFFF)) % u32(T))
        r = (r % u32(T)).astype(jnp.int32)

    tblT = jax.lax.dot_general(se_ref[:], op_ref[:], (((1,), (1,)), ((), ())),
                               preferred_element_type=f32)
    tmax = jnp.max(tblT, axis=0, keepdims=True)
    lse = jnp.log(jnp.sum(jnp.exp(tblT - tmax), axis=0, keepdims=True))
    aug = jnp.concatenate([tblT, tmax + lse], axis=0)

    io_s = jax.lax.broadcasted_iota(jnp.int32, (S, _BB), 0)
    sio = jax.lax.broadcasted_iota(u32, (S, _BB), 0)
    cbase = u32(L * S) * bvec
    acc = lp_num
    ind_cols, sc_cols = [], []
    for l in range(L):
        mask_l = nt <= l
        ind_l = jnp.where(mask_l, 0, r[l:l + 1, :])
        ind_cols.append(ind_l)
        lvl = [aug[:, k:k + 1] for k in range(T)]
        bit = 1
        while len(lvl) > 1:
            cond = (ind_l & bit) != 0
            lvl = [jnp.where(cond, lvl[2 * j + 1], lvl[2 * j])
                   for j in range(len(lvl) // 2)]
            bit <<= 1
        rows = lvl[0][:S, :]
        logz = lvl[0][S:, :]
        gc_l = _gumbel(_KC, cbase + u32(S * l) + sio)
        y = rows + gc_l
        ym = jnp.max(y, axis=0, keepdims=True)
        sc = jnp.min(jnp.where(y == ym, io_s, S), axis=0, keepdims=True)
        sc_cols.append(sc)
        chosen = jnp.sum(jnp.where(io_s == sc, rows, 0.0), axis=0, keepdims=True)
        acc = acc + jnp.where(mask_l, 0.0, chosen - logz)
    inds_ref[:, :] = jnp.concatenate(ind_cols, axis=0).T
    sc_ref[:, :] = jnp.concatenate(sc_cols, axis=0).T
    lp_ref[:, :] = acc.T


def kernel(imgs, op_embs, num_transforms_embs, scale_embs, q, pnst):
    B = imgs.shape[0]
    T = op_embs.shape[0]
    S = scale_embs.shape[0]
    NH = num_transforms_embs.shape[0]
    L = NH - 1
    H = q.shape[0]

    q_c = q.reshape(H, 1)
    pnst_c = pnst.reshape(NH, 1)

    nblk = B // _BB
    full = lambda *shape: pl.BlockSpec(shape, lambda i: (0,) * len(shape))
    inds, sc, lp = pl.pallas_call(
        functools.partial(_body, L=L, T=T, S=S, NH=NH),
        grid=(nblk,),
        in_specs=[
            full(T, H),
            full(NH, H),
            full(S, H),
            full(H, 1),
            full(NH, 1),
        ],
        out_specs=[
            pl.BlockSpec((_BB, L), lambda i: (i, 0)),
            pl.BlockSpec((_BB, L), lambda i: (i, 0)),
            pl.BlockSpec((_BB, 1), lambda i: (i, 0)),
        ],
        out_shape=[
            jax.ShapeDtypeStruct((B, L), jnp.int32),
            jax.ShapeDtypeStruct((B, L), jnp.int32),
            jax.ShapeDtypeStruct((B, 1), jnp.float32),
        ],
    )(op_embs, num_transforms_embs, scale_embs, q_c, pnst_c)

    return (inds, sc, lp.reshape(B))
